# Initial kernel scaffold; baseline (speedup 1.0000x reference)
import jax
import jax.numpy as jnp
from jax.experimental import pallas as pl


def kernel(user_id, item_id, rating, preds, y_u_t, y_v):
    raise NotImplementedError("write your pallas kernel here")


# speedup vs baseline: 15.9047x; 15.9047x over previous
import functools

import jax
import jax.numpy as jnp
from jax import lax
from jax.experimental import pallas as pl
from jax.experimental.pallas import tpu as pltpu
from jax.experimental.pallas import tpu_sc as plsc

_N = 819200
_NUM_USERS = 16384
_NC = 2
_NS = 16
_NW = _NC * _NS
_CHUNK = _N // _NW
_VECS = _CHUNK // 16
_TBL_VECS = _NUM_USERS // 16

_DENSE_GRID = 16
_BCE_ROWS = _N // 128 // _DENSE_GRID
_YV_PAD_ROWS = 51200
_YV_ROWS = _YV_PAD_ROWS // _DENSE_GRID
_CMB_GRID = 8
_CMB_COLS = _NUM_USERS // _CMB_GRID


def _sc_segment_body(uid_hbm, preds_hbm, yut_hbm,
                     out_min, out_sum, out_cnt,
                     ids_v, p_v, t_v, min_t, sum_t, cnt_t,
                     scr_i, scr_nx, scr_p, scr_y, scr_c, sem):
    wid = lax.axis_index("s") * _NC + lax.axis_index("c")
    base = wid * _CHUNK
    cp0 = pltpu.async_copy(uid_hbm.at[pl.ds(base, _CHUNK)], ids_v, sem)
    cp1 = pltpu.async_copy(preds_hbm.at[pl.ds(base, _CHUNK)], p_v, sem)
    cp2 = pltpu.async_copy(yut_hbm.at[pl.ds(base, _CHUNK)], t_v, sem)

    inf16 = jnp.full((16,), jnp.inf, jnp.float32)
    zero16 = jnp.zeros((16,), jnp.float32)
    neg16 = jnp.full((16,), -1, jnp.int32)

    def init_body(i, c):
        off = i * 16
        min_t[pl.ds(off, 16)] = inf16
        sum_t[pl.ds(off, 16)] = zero16
        cnt_t[pl.ds(off, 16)] = zero16
        return c

    lax.fori_loop(0, _TBL_VECS, init_body, 0)

    scr_i[pl.ds(0, 16)] = neg16
    scr_nx[pl.ds(16, 16)] = neg16
    scr_p[pl.ds(0, 16)] = zero16
    scr_y[pl.ds(0, 16)] = zero16
    scr_c[pl.ds(0, 16)] = zero16

    cp0.wait()
    cp1.wait()
    cp2.wait()

    ones16 = jnp.ones((16,), jnp.float32)

    def body(j, c):
        off = j * 16
        ids = ids_v[pl.ds(off, 16)]
        p = p_v[pl.ds(off, 16)]
        yt = t_v[pl.ds(off, 16)]
        scr_i[pl.ds(16, 16)] = ids
        scr_nx[pl.ds(0, 16)] = ids
        p_run, y_run, c_run = p, yt, ones16
        for k in (1, 2, 4, 8):
            scr_p[pl.ds(16, 16)] = p_run
            scr_y[pl.ds(16, 16)] = y_run
            scr_c[pl.ds(16, 16)] = c_run
            same = scr_i[pl.ds(16 - k, 16)] == ids
            p_run = jnp.where(same, jnp.minimum(p_run, scr_p[pl.ds(16 - k, 16)]), p_run)
            y_run = jnp.where(same, y_run + scr_y[pl.ds(16 - k, 16)], y_run)
            c_run = jnp.where(same, c_run + scr_c[pl.ds(16 - k, 16)], c_run)
        is_last = ids != scr_nx[pl.ds(1, 16)]
        cur = plsc.load_gather(min_t, [ids])
        plsc.store_scatter(min_t, [ids], jnp.minimum(cur, p_run), mask=is_last)
        plsc.addupdate_scatter(sum_t, [ids], y_run, mask=is_last)
        plsc.addupdate_scatter(cnt_t, [ids], c_run, mask=is_last)
        return c

    lax.fori_loop(0, _VECS, body, 0)

    pltpu.sync_copy(min_t, out_min.at[wid])
    pltpu.sync_copy(sum_t, out_sum.at[wid])
    pltpu.sync_copy(cnt_t, out_cnt.at[wid])


_sc_segment = functools.partial(
    pl.kernel,
    mesh=plsc.VectorSubcoreMesh(core_axis_name="c", subcore_axis_name="s"),
    compiler_params=pltpu.CompilerParams(needs_layout_passes=False),
    out_type=[
        jax.ShapeDtypeStruct((_NW, _NUM_USERS), jnp.float32),
        jax.ShapeDtypeStruct((_NW, _NUM_USERS), jnp.float32),
        jax.ShapeDtypeStruct((_NW, _NUM_USERS), jnp.float32),
    ],
    scratch_types=[
        pltpu.VMEM((_CHUNK,), jnp.int32),
        pltpu.VMEM((_CHUNK,), jnp.float32),
        pltpu.VMEM((_CHUNK,), jnp.float32),
        pltpu.VMEM((_NUM_USERS,), jnp.float32),
        pltpu.VMEM((_NUM_USERS,), jnp.float32),
        pltpu.VMEM((_NUM_USERS,), jnp.float32),
        pltpu.VMEM((32,), jnp.int32),
        pltpu.VMEM((32,), jnp.int32),
        pltpu.VMEM((32,), jnp.float32),
        pltpu.VMEM((32,), jnp.float32),
        pltpu.VMEM((32,), jnp.float32),
        pltpu.SemaphoreType.DMA,
    ],
)(_sc_segment_body)


def _tc_dense_body(p_ref, r_ref, v_ref, o_ref):
    @pl.when(pl.program_id(0) == 0)
    def _init():
        o_ref[0, 0] = 0.0

    p = p_ref[...]
    r = r_ref[...]
    lp = jnp.maximum(jnp.log(p), -100.0)
    l1 = jnp.maximum(jnp.log1p(-p), -100.0)
    bce = jnp.sum(r * lp + (1.0 - r) * l1)
    v = v_ref[...]
    o_ref[0, 0] += jnp.sum(v * v) - (1000.0 / _N) * bce


_tc_dense = pl.pallas_call(
    _tc_dense_body,
    grid=(_DENSE_GRID,),
    in_specs=[
        pl.BlockSpec((_BCE_ROWS, 128), lambda i: (i, 0)),
        pl.BlockSpec((_BCE_ROWS, 128), lambda i: (i, 0)),
        pl.BlockSpec((_YV_ROWS, 128), lambda i: (i, 0)),
    ],
    out_specs=pl.BlockSpec((1, 1), lambda i: (0, 0), memory_space=pltpu.SMEM),
    out_shape=jax.ShapeDtypeStruct((1, 1), jnp.float32),
)


def _tc_combine_body(m_ref, s_ref, c_ref, o_ref):
    @pl.when(pl.program_id(0) == 0)
    def _init():
        o_ref[0, 0] = 0.0

    m = jnp.min(m_ref[...], axis=0, keepdims=True)
    s = jnp.sum(s_ref[...], axis=0, keepdims=True)
    c = jnp.sum(c_ref[...], axis=0, keepdims=True)
    l_u = jnp.maximum(jnp.log(m) - jnp.log(s / c), -1.0)
    o_ref[0, 0] += jnp.sum(l_u)


_tc_combine = pl.pallas_call(
    _tc_combine_body,
    grid=(_CMB_GRID,),
    in_specs=[
        pl.BlockSpec((_NW, _CMB_COLS), lambda i: (0, i)),
        pl.BlockSpec((_NW, _CMB_COLS), lambda i: (0, i)),
        pl.BlockSpec((_NW, _CMB_COLS), lambda i: (0, i)),
    ],
    out_specs=pl.BlockSpec((1, 1), lambda i: (0, 0), memory_space=pltpu.SMEM),
    out_shape=jax.ShapeDtypeStruct((1, 1), jnp.float32),
)


def kernel(user_id, item_id, rating, preds, y_u_t, y_v):
    del item_id
    uid = user_id.astype(jnp.int32)
    mins, sums, cnts = _sc_segment(uid, preds, y_u_t)
    yv_flat = y_v.reshape(-1, 128)
    yv_pad = jnp.concatenate(
        [yv_flat, jnp.zeros((_YV_PAD_ROWS - yv_flat.shape[0], 128), jnp.float32)]
    )
    dense = _tc_dense(preds.reshape(-1, 128), rating.reshape(-1, 128), yv_pad)
    lu = _tc_combine(mins, sums, cnts)
    return lu[0, 0] + dense[0, 0]
